# Initial kernel scaffold; baseline (speedup 1.0000x reference)
import jax
import jax.numpy as jnp
from jax.experimental import pallas as pl


def kernel(x, edge_index, edge_attr, batch, l0_w1, l0_b1, l0_w2, l0_b2, l0_root, l0_bias, l1_w1, l1_b1, l1_w2, l1_b2, l1_root, l1_bias, fc1_w, fc1_b, fc2_w, fc2_b, fc3_w, fc3_b):
    raise NotImplementedError("write your pallas kernel here")


# speedup vs baseline: 1.2970x; 1.2970x over previous
import functools

import jax
import jax.numpy as jnp
from jax import lax
from jax.experimental import pallas as pl
from jax.experimental.pallas import tpu as pltpu
from jax.experimental.pallas import tpu_sc as plsc

NC = 2
NS = 16
NW = NC * NS


def _elu(v):
    return jnp.where(v > 0, v, jnp.exp(v) - 1.0)


def _sc_gather(table, idx):
    n_rows, d = table.shape
    e = idx.shape[0]
    per_w = e // NW
    assert per_w * NW == e and per_w % 8 == 0
    chunk = 128
    n_full = per_w // chunk
    tail = per_w - n_full * chunk

    mesh = plsc.VectorSubcoreMesh(core_axis_name="c", subcore_axis_name="s")

    @functools.partial(
        pl.kernel,
        out_type=jax.ShapeDtypeStruct((e, d), table.dtype),
        mesh=mesh,
        scratch_types=[
            pltpu.VMEM((per_w,), jnp.int32),
            pltpu.VMEM((chunk, d), table.dtype),
            pltpu.SemaphoreType.DMA,
        ],
    )
    def k(table_hbm, idx_hbm, out_hbm, idx_v, rows_v, sem):
        wid = lax.axis_index("s") * NC + lax.axis_index("c")
        base = wid * per_w
        pltpu.sync_copy(idx_hbm.at[pl.ds(base, per_w)], idx_v)

        @pl.loop(0, n_full)
        def _(j):
            pltpu.async_copy(
                table_hbm.at[idx_v.at[pl.ds(j * chunk, chunk)]], rows_v, sem
            ).wait()
            pltpu.sync_copy(rows_v, out_hbm.at[pl.ds(base + j * chunk, chunk)])

        if tail:
            pltpu.async_copy(
                table_hbm.at[idx_v.at[pl.ds(n_full * chunk, tail)]],
                rows_v.at[pl.ds(0, tail)], sem,
            ).wait()
            pltpu.sync_copy(
                rows_v.at[pl.ds(0, tail)],
                out_hbm.at[pl.ds(base + n_full * chunk, tail)],
            )

    return k(table, idx)


def _sc_scatter_add(msg, dst3, zeros_nd):
    e, d = msg.shape
    n_nodes = zeros_nd.shape[0]
    nw, n_chunk, chunk = dst3.shape
    assert nw == NW and n_chunk * chunk * NW == e and chunk % 8 == 0
    per_w = n_chunk * chunk
    stripe = (n_nodes // NS) // 8 * 8
    rem = n_nodes - stripe * NS
    assert rem % 8 == 0

    mesh = plsc.VectorSubcoreMesh(core_axis_name="c", subcore_axis_name="s")

    @functools.partial(
        pl.kernel,
        out_type=jax.ShapeDtypeStruct((NC, n_nodes, d), msg.dtype),
        mesh=mesh,
        scratch_types=[
            pltpu.VMEM((n_chunk, chunk), jnp.int32),
            pltpu.VMEM((chunk, d), msg.dtype),
            pltpu.VMEM_SHARED((n_nodes, d), msg.dtype),
        ],
    )
    def k(msg_hbm, dst_hbm, zeros_hbm, out_hbm, idx_v, msg_v, acc_sh):
        cid = lax.axis_index("c")
        sid = lax.axis_index("s")
        wid = sid * NC + cid
        base = wid * per_w

        pltpu.sync_copy(
            zeros_hbm.at[pl.ds(sid * stripe, stripe)],
            acc_sh.at[pl.ds(sid * stripe, stripe)],
        )
        if rem:
            @pl.when(sid == NS - 1)
            def _():
                pltpu.sync_copy(
                    zeros_hbm.at[pl.ds(NS * stripe, rem)],
                    acc_sh.at[pl.ds(NS * stripe, rem)],
                )
        pltpu.sync_copy(dst_hbm.at[wid], idx_v)
        plsc.subcore_barrier()

        @pl.loop(0, n_chunk)
        def _(j):
            pltpu.sync_copy(msg_hbm.at[pl.ds(base + j * chunk, chunk)], msg_v)
            pltpu.sync_copy(msg_v, acc_sh.at[idx_v.at[j]], add=True)

        plsc.subcore_barrier()
        pltpu.sync_copy(
            acc_sh.at[pl.ds(sid * stripe, stripe)],
            out_hbm.at[cid, pl.ds(sid * stripe, stripe)],
        )
        if rem:
            @pl.when(sid == NS - 1)
            def _():
                pltpu.sync_copy(
                    acc_sh.at[pl.ds(NS * stripe, rem)],
                    out_hbm.at[cid, pl.ds(NS * stripe, rem)],
                )

    return k(msg, dst3, zeros_nd)


EB = 256


def _c0_body(ea_ref, gx_ref, w1_ref, b1_ref, w2p_ref, b2p_ref, out_ref):
    h = jnp.maximum(
        jnp.dot(ea_ref[...], w1_ref[...], preferred_element_type=jnp.float32)
        + b1_ref[...], 0.0)
    wp = jnp.dot(h.astype(jnp.bfloat16), w2p_ref[...],
                 preferred_element_type=jnp.float32) + b2p_ref[...]
    gx = gx_ref[...]
    d_out = w2p_ref.shape[1] // 128
    cols = []
    for o in range(d_out):
        blk = wp[:, o * 128:(o + 1) * 128]
        cols.append(jnp.sum(blk * gx, axis=1, keepdims=True))
    cols.append(jnp.zeros((gx.shape[0], 128 - d_out), jnp.float32))
    out_ref[...] = jnp.concatenate(cols, axis=1)


def _c0(ea, gx, w1, b1, w2p, b2p):
    e = ea.shape[0]
    grid = (e // EB,)
    return pl.pallas_call(
        _c0_body,
        grid=grid,
        in_specs=[
            pl.BlockSpec((EB, ea.shape[1]), lambda i: (i, 0)),
            pl.BlockSpec((EB, gx.shape[1]), lambda i: (i, 0)),
            pl.BlockSpec(w1.shape, lambda i: (0, 0)),
            pl.BlockSpec(b1.shape, lambda i: (0, 0)),
            pl.BlockSpec(w2p.shape, lambda i: (0, 0)),
            pl.BlockSpec(b2p.shape, lambda i: (0, 0)),
        ],
        out_specs=pl.BlockSpec((EB, 128), lambda i: (i, 0)),
        out_shape=jax.ShapeDtypeStruct((e, 128), jnp.float32),
    )(ea, gx, w1, b1, w2p, b2p)


def _c1_body(ea_ref, gh_ref, w1_ref, b1_ref, w2_ref, b2_ref, out_ref):
    d_in = gh_ref.shape[1] // 4
    d_out = w2_ref.shape[1] // d_in
    h = jnp.maximum(
        jnp.dot(ea_ref[...], w1_ref[...], preferred_element_type=jnp.float32)
        + b1_ref[...], 0.0)
    w = jnp.dot(h.astype(jnp.bfloat16), w2_ref[...],
                preferred_element_type=jnp.float32) + b2_ref[...]
    gh = gh_ref[...][:, :d_in]
    acc = jnp.zeros((gh.shape[0], d_out), jnp.float32)
    for i in range(d_in):
        acc = acc + gh[:, i:i + 1] * w[:, i * d_out:(i + 1) * d_out]
    out_ref[...] = jnp.concatenate(
        [acc, jnp.zeros((acc.shape[0], 128 - d_out), jnp.float32)], axis=1)


def _c1(ea, gh, w1, b1, w2, b2):
    e = ea.shape[0]
    grid = (e // EB,)
    return pl.pallas_call(
        _c1_body,
        grid=grid,
        in_specs=[
            pl.BlockSpec((EB, ea.shape[1]), lambda i: (i, 0)),
            pl.BlockSpec((EB, gh.shape[1]), lambda i: (i, 0)),
            pl.BlockSpec(w1.shape, lambda i: (0, 0)),
            pl.BlockSpec(b1.shape, lambda i: (0, 0)),
            pl.BlockSpec(w2.shape, lambda i: (0, 0)),
            pl.BlockSpec(b2.shape, lambda i: (0, 0)),
        ],
        out_specs=pl.BlockSpec((EB, 128), lambda i: (i, 0)),
        out_shape=jax.ShapeDtypeStruct((e, 128), jnp.float32),
    )(ea, gh, w1, b1, w2, b2)


def _r_body(p_ref, x_ref, root_ref, bias_ref, out_ref):
    d = root_ref.shape[1]
    agg = p_ref[0][:, :d] + p_ref[1][:, :d]
    xr = jnp.dot(x_ref[...], root_ref[...], preferred_element_type=jnp.float32)
    h = _elu(agg + xr + bias_ref[...])
    pad = out_ref.shape[1] - h.shape[1]
    out_ref[...] = jnp.concatenate(
        [h, jnp.zeros((h.shape[0], pad), jnp.float32)], axis=1)


def _r(parts, x, root, bias):
    n = parts.shape[1]
    return pl.pallas_call(
        _r_body,
        in_specs=[
            pl.BlockSpec(parts.shape, lambda: (0, 0, 0)),
            pl.BlockSpec(x.shape, lambda: (0, 0)),
            pl.BlockSpec(root.shape, lambda: (0, 0)),
            pl.BlockSpec(bias.shape, lambda: (0, 0)),
        ],
        out_specs=pl.BlockSpec((n, 128), lambda: (0, 0)),
        out_shape=jax.ShapeDtypeStruct((n, 128), jnp.float32),
    )(parts, x, root, bias)


def _readout_body(p_ref, h0_ref, root_ref, bias_ref, batch_ref,
                  fc1w_ref, fc1b_ref, fc2w_ref, fc2b_ref, fc3w_ref, fc3b_ref,
                  out_ref):
    n = h0_ref.shape[0]
    g_count = out_ref.shape[0]
    d = root_ref.shape[1]
    h1 = _elu(p_ref[0][:, :d] + p_ref[1][:, :d]
              + jnp.dot(h0_ref[...][:, :root_ref.shape[0]], root_ref[...],
                        preferred_element_type=jnp.float32)
              + bias_ref[...])
    gids = lax.broadcasted_iota(jnp.int32, (n, g_count), 1)
    oneh = (batch_ref[...] == gids).astype(jnp.float32)
    gsum = lax.dot_general(oneh, h1, (((0,), (0,)), ((), ())),
                           preferred_element_type=jnp.float32)
    counts = jnp.sum(oneh, axis=0)
    g = gsum / jnp.clip(counts, 1.0)[:, None]
    g = _elu(jnp.dot(g, fc1w_ref[...], preferred_element_type=jnp.float32)
             + fc1b_ref[...])
    g = _elu(jnp.dot(g, fc2w_ref[...], preferred_element_type=jnp.float32)
             + fc2b_ref[...])
    out_ref[...] = (jnp.dot(g, fc3w_ref[...],
                            preferred_element_type=jnp.float32)
                    + fc3b_ref[...])


def _readout(parts, h0, root, bias, batch2, n_graphs,
             fc1w, fc1b, fc2w, fc2b, fc3w, fc3b):
    n = h0.shape[0]
    full = lambda a: pl.BlockSpec(a.shape, lambda: tuple(0 for _ in a.shape))
    out = pl.pallas_call(
        _readout_body,
        in_specs=[full(parts), full(h0), full(root), full(bias), full(batch2),
                  full(fc1w), full(fc1b), full(fc2w), full(fc2b), full(fc3w),
                  full(fc3b)],
        out_specs=pl.BlockSpec((n_graphs, 1), lambda: (0, 0)),
        out_shape=jax.ShapeDtypeStruct((n_graphs, 1), jnp.float32),
    )(parts, h0, root, bias, batch2, fc1w, fc1b, fc2w, fc2b, fc3w, fc3b)
    return out.reshape(-1)


def kernel(x, edge_index, edge_attr, batch,
           l0_w1, l0_b1, l0_w2, l0_b2, l0_root, l0_bias,
           l1_w1, l1_b1, l1_w2, l1_b2, l1_root, l1_bias,
           fc1_w, fc1_b, fc2_w, fc2_b, fc3_w, fc3_b):
    n, d = x.shape
    e = edge_attr.shape[0]
    d0 = l0_root.shape[1]
    d1 = l1_root.shape[1]
    n_graphs = 64

    src = edge_index[0]
    dst = edge_index[1]
    per_w = e // NW
    chunk = 40
    n_chunk = per_w // chunk
    dst3 = dst.reshape(NW, n_chunk, chunk)

    w2p0 = l0_w2.reshape(d, d, d0).transpose(0, 2, 1).reshape(d, d * d0)
    b2p0 = l0_b2.reshape(d, d0).T.reshape(1, d * d0)
    w2p0 = w2p0.astype(jnp.bfloat16)
    w2_1 = l1_w2.astype(jnp.bfloat16)
    b2_1 = l1_b2.reshape(1, d0 * d1)
    b1_0 = l0_b1.reshape(1, -1)
    b1_1 = l1_b1.reshape(1, -1)
    bias0 = l0_bias.reshape(1, -1)
    bias1 = l1_bias.reshape(1, -1)

    zeros128 = jnp.zeros((n, 128), jnp.float32)

    gx = _sc_gather(x, src)
    msg0 = _c0(edge_attr, gx, l0_w1, b1_0, w2p0, b2p0)
    parts0 = _sc_scatter_add(msg0, dst3, zeros128)
    h0 = _r(parts0, x, l0_root, bias0)
    gh = _sc_gather(h0, src)
    msg1 = _c1(edge_attr, gh, l1_w1, b1_1, w2_1, b2_1)
    parts1 = _sc_scatter_add(msg1, dst3, zeros128)
    out = _readout(parts1, h0, l1_root, bias1,
                   batch.reshape(n, 1), n_graphs,
                   fc1_w, fc1_b.reshape(1, -1),
                   fc2_w, fc2_b.reshape(1, -1),
                   fc3_w, fc3_b.reshape(1, -1))
    return out


# speedup vs baseline: 1.7036x; 1.3135x over previous
import functools

import jax
import jax.numpy as jnp
from jax import lax
from jax.experimental import pallas as pl
from jax.experimental.pallas import tpu as pltpu
from jax.experimental.pallas import tpu_sc as plsc

NC = 2
NS = 16
NW = NC * NS


def _elu(v):
    return jnp.where(v > 0, v, jnp.exp(v) - 1.0)


def _sc_gather(table, idx):
    n_rows, d = table.shape
    e = idx.shape[0]
    per_w = e // NW
    assert per_w * NW == e and per_w % 8 == 0
    chunk = 128
    n_full = per_w // chunk
    tail = per_w - n_full * chunk

    mesh = plsc.VectorSubcoreMesh(core_axis_name="c", subcore_axis_name="s")

    @functools.partial(
        pl.kernel,
        out_type=jax.ShapeDtypeStruct((e, d), table.dtype),
        mesh=mesh,
        scratch_types=[
            pltpu.VMEM((per_w,), jnp.int32),
            pltpu.VMEM((chunk, d), table.dtype),
            pltpu.SemaphoreType.DMA,
        ],
    )
    def k(table_hbm, idx_hbm, out_hbm, idx_v, rows_v, sem):
        wid = lax.axis_index("s") * NC + lax.axis_index("c")
        base = wid * per_w
        pltpu.sync_copy(idx_hbm.at[pl.ds(base, per_w)], idx_v)

        @pl.loop(0, n_full)
        def _(j):
            pltpu.async_copy(
                table_hbm.at[idx_v.at[pl.ds(j * chunk, chunk)]], rows_v, sem
            ).wait()
            pltpu.sync_copy(rows_v, out_hbm.at[pl.ds(base + j * chunk, chunk)])

        if tail:
            pltpu.async_copy(
                table_hbm.at[idx_v.at[pl.ds(n_full * chunk, tail)]],
                rows_v.at[pl.ds(0, tail)], sem,
            ).wait()
            pltpu.sync_copy(
                rows_v.at[pl.ds(0, tail)],
                out_hbm.at[pl.ds(base + n_full * chunk, tail)],
            )

    return k(table, idx)


def _sc_scatter_add(msg, dst3, zeros_nd):
    e, d = msg.shape
    n_nodes = zeros_nd.shape[0]
    nw, n_chunk, chunk = dst3.shape
    assert nw == NW and n_chunk * chunk * NW == e and chunk % 8 == 0
    per_w = n_chunk * chunk
    stripe = (n_nodes // NS) // 8 * 8
    rem = n_nodes - stripe * NS
    assert rem % 8 == 0

    mesh = plsc.VectorSubcoreMesh(core_axis_name="c", subcore_axis_name="s")

    @functools.partial(
        pl.kernel,
        out_type=jax.ShapeDtypeStruct((NC, n_nodes, d), msg.dtype),
        mesh=mesh,
        scratch_types=[
            pltpu.VMEM((n_chunk, chunk), jnp.int32),
            pltpu.VMEM((chunk, d), msg.dtype),
            pltpu.VMEM_SHARED((n_nodes, d), msg.dtype),
        ],
    )
    def k(msg_hbm, dst_hbm, zeros_hbm, out_hbm, idx_v, msg_v, acc_sh):
        cid = lax.axis_index("c")
        sid = lax.axis_index("s")
        wid = sid * NC + cid
        base = wid * per_w

        pltpu.sync_copy(
            zeros_hbm.at[pl.ds(sid * stripe, stripe)],
            acc_sh.at[pl.ds(sid * stripe, stripe)],
        )
        if rem:
            @pl.when(sid == NS - 1)
            def _():
                pltpu.sync_copy(
                    zeros_hbm.at[pl.ds(NS * stripe, rem)],
                    acc_sh.at[pl.ds(NS * stripe, rem)],
                )
        pltpu.sync_copy(dst_hbm.at[wid], idx_v)
        plsc.subcore_barrier()

        @pl.loop(0, n_chunk)
        def _(j):
            pltpu.sync_copy(msg_hbm.at[pl.ds(base + j * chunk, chunk)], msg_v)
            pltpu.sync_copy(msg_v, acc_sh.at[idx_v.at[j]], add=True)

        plsc.subcore_barrier()
        pltpu.sync_copy(
            acc_sh.at[pl.ds(sid * stripe, stripe)],
            out_hbm.at[cid, pl.ds(sid * stripe, stripe)],
        )
        if rem:
            @pl.when(sid == NS - 1)
            def _():
                pltpu.sync_copy(
                    acc_sh.at[pl.ds(NS * stripe, rem)],
                    out_hbm.at[cid, pl.ds(NS * stripe, rem)],
                )

    return k(msg, dst3, zeros_nd)


EB = 256


def _expand(v, r_ref):
    hi = v.astype(jnp.bfloat16)
    lo = (v - hi.astype(jnp.float32)).astype(jnp.bfloat16)
    r = r_ref[...]
    return (jnp.dot(hi, r, preferred_element_type=jnp.float32)
            + jnp.dot(lo, r, preferred_element_type=jnp.float32))


def _fold_contract(wp, vr, width, d_out):
    p = wp * vr
    while width > d_out:
        width //= 2
        p = p[:, :width] + p[:, width:2 * width]
    return p


def _c0_body(ea_ref, gx_ref, w1_ref, b1_ref, w2p_ref, b2m_ref, out_ref):
    h = jnp.maximum(
        jnp.dot(ea_ref[...], w1_ref[...], preferred_element_type=jnp.float32)
        + b1_ref[...], 0.0)
    wp = jnp.dot(h.astype(jnp.bfloat16), w2p_ref[...],
                 preferred_element_type=jnp.float32)
    gx = gx_ref[...]
    d_out = w2p_ref.shape[1] // 128
    cols = []
    for o in range(d_out):
        blk = wp[:, o * 128:(o + 1) * 128]
        cols.append(jnp.sum(blk * gx, axis=1, keepdims=True))
    p = jnp.concatenate(cols, axis=1) + jnp.dot(
        gx, b2m_ref[...], preferred_element_type=jnp.float32)
    out_ref[:, :d_out] = p


def _c0(ea, gx, w1, b1, w2p, b2m):
    e = ea.shape[0]
    grid = (e // EB,)
    return pl.pallas_call(
        _c0_body,
        grid=grid,
        in_specs=[
            pl.BlockSpec((EB, ea.shape[1]), lambda i: (i, 0)),
            pl.BlockSpec((EB, gx.shape[1]), lambda i: (i, 0)),
            pl.BlockSpec(w1.shape, lambda i: (0, 0)),
            pl.BlockSpec(b1.shape, lambda i: (0, 0)),
            pl.BlockSpec(w2p.shape, lambda i: (0, 0)),
            pl.BlockSpec(b2m.shape, lambda i: (0, 0)),
        ],
        out_specs=pl.BlockSpec((EB, 128), lambda i: (i, 0)),
        out_shape=jax.ShapeDtypeStruct((e, 128), jnp.float32),
    )(ea, gx, w1, b1, w2p, b2m)


def _c1_body(ea_ref, gh_ref, w1_ref, b1_ref, w2_ref, r_ref, b2m_ref, out_ref):
    d_in = r_ref.shape[0]
    d_out = w2_ref.shape[1] // d_in
    h = jnp.maximum(
        jnp.dot(ea_ref[...], w1_ref[...], preferred_element_type=jnp.float32)
        + b1_ref[...], 0.0)
    w = jnp.dot(h.astype(jnp.bfloat16), w2_ref[...],
                preferred_element_type=jnp.float32)
    gh = gh_ref[...][:, :d_in]
    ghr = _expand(gh, r_ref)
    p = _fold_contract(w, ghr, d_in * d_out, d_out)
    p = p + jnp.dot(gh, b2m_ref[...], preferred_element_type=jnp.float32)
    out_ref[:, :d_out] = p


def _c1(ea, gh, w1, b1, w2, rmat, b2m):
    e = ea.shape[0]
    grid = (e // EB,)
    return pl.pallas_call(
        _c1_body,
        grid=grid,
        in_specs=[
            pl.BlockSpec((EB, ea.shape[1]), lambda i: (i, 0)),
            pl.BlockSpec((EB, gh.shape[1]), lambda i: (i, 0)),
            pl.BlockSpec(w1.shape, lambda i: (0, 0)),
            pl.BlockSpec(b1.shape, lambda i: (0, 0)),
            pl.BlockSpec(w2.shape, lambda i: (0, 0)),
            pl.BlockSpec(rmat.shape, lambda i: (0, 0)),
            pl.BlockSpec(b2m.shape, lambda i: (0, 0)),
        ],
        out_specs=pl.BlockSpec((EB, 128), lambda i: (i, 0)),
        out_shape=jax.ShapeDtypeStruct((e, 128), jnp.float32),
    )(ea, gh, w1, b1, w2, rmat, b2m)


def _r_body(p_ref, x_ref, root_ref, bias_ref, out_ref):
    d = root_ref.shape[1]
    agg = p_ref[0][:, :d] + p_ref[1][:, :d]
    xr = jnp.dot(x_ref[...], root_ref[...], preferred_element_type=jnp.float32)
    h = _elu(agg + xr + bias_ref[...])
    pad = out_ref.shape[1] - h.shape[1]
    out_ref[...] = jnp.concatenate(
        [h, jnp.zeros((h.shape[0], pad), jnp.float32)], axis=1)


def _r(parts, x, root, bias):
    n = parts.shape[1]
    return pl.pallas_call(
        _r_body,
        in_specs=[
            pl.BlockSpec(parts.shape, lambda: (0, 0, 0)),
            pl.BlockSpec(x.shape, lambda: (0, 0)),
            pl.BlockSpec(root.shape, lambda: (0, 0)),
            pl.BlockSpec(bias.shape, lambda: (0, 0)),
        ],
        out_specs=pl.BlockSpec((n, 128), lambda: (0, 0)),
        out_shape=jax.ShapeDtypeStruct((n, 128), jnp.float32),
    )(parts, x, root, bias)


def _readout_body(p_ref, h0_ref, root_ref, bias_ref, batch_ref,
                  fc1w_ref, fc1b_ref, fc2w_ref, fc2b_ref, fc3w_ref, fc3b_ref,
                  out_ref):
    n = h0_ref.shape[0]
    g_count = out_ref.shape[0]
    d = root_ref.shape[1]
    h1 = _elu(p_ref[0][:, :d] + p_ref[1][:, :d]
              + jnp.dot(h0_ref[...][:, :root_ref.shape[0]], root_ref[...],
                        preferred_element_type=jnp.float32)
              + bias_ref[...])
    gids = lax.broadcasted_iota(jnp.int32, (n, g_count), 1)
    oneh = (batch_ref[...] == gids).astype(jnp.float32)
    gsum = lax.dot_general(oneh, h1, (((0,), (0,)), ((), ())),
                           preferred_element_type=jnp.float32)
    counts = jnp.sum(oneh, axis=0)
    g = gsum / jnp.clip(counts, 1.0)[:, None]
    g = _elu(jnp.dot(g, fc1w_ref[...], preferred_element_type=jnp.float32)
             + fc1b_ref[...])
    g = _elu(jnp.dot(g, fc2w_ref[...], preferred_element_type=jnp.float32)
             + fc2b_ref[...])
    out_ref[...] = (jnp.dot(g, fc3w_ref[...],
                            preferred_element_type=jnp.float32)
                    + fc3b_ref[...])


def _readout(parts, h0, root, bias, batch2, n_graphs,
             fc1w, fc1b, fc2w, fc2b, fc3w, fc3b):
    n = h0.shape[0]
    full = lambda a: pl.BlockSpec(a.shape, lambda: tuple(0 for _ in a.shape))
    out = pl.pallas_call(
        _readout_body,
        in_specs=[full(parts), full(h0), full(root), full(bias), full(batch2),
                  full(fc1w), full(fc1b), full(fc2w), full(fc2b), full(fc3w),
                  full(fc3b)],
        out_specs=pl.BlockSpec((n_graphs, 1), lambda: (0, 0)),
        out_shape=jax.ShapeDtypeStruct((n_graphs, 1), jnp.float32),
    )(parts, h0, root, bias, batch2, fc1w, fc1b, fc2w, fc2b, fc3w, fc3b)
    return out.reshape(-1)


def kernel(x, edge_index, edge_attr, batch,
           l0_w1, l0_b1, l0_w2, l0_b2, l0_root, l0_bias,
           l1_w1, l1_b1, l1_w2, l1_b2, l1_root, l1_bias,
           fc1_w, fc1_b, fc2_w, fc2_b, fc3_w, fc3_b):
    n, d = x.shape
    e = edge_attr.shape[0]
    d0 = l0_root.shape[1]
    d1 = l1_root.shape[1]
    n_graphs = 64

    src = edge_index[0]
    dst = edge_index[1]
    per_w = e // NW
    chunk = 40
    n_chunk = per_w // chunk
    dst3 = dst.reshape(NW, n_chunk, chunk)

    w2p0 = l0_w2.reshape(d, d, d0).transpose(0, 2, 1).reshape(
        d, d * d0).astype(jnp.bfloat16)
    b2m0 = l0_b2.reshape(d, d0)
    w2_1 = l1_w2.astype(jnp.bfloat16)
    b2m1 = l1_b2.reshape(d0, d1)
    r1 = jnp.kron(jnp.eye(d0, dtype=jnp.float32),
                  jnp.ones((1, d1), jnp.float32)).astype(jnp.bfloat16)
    b1_0 = l0_b1.reshape(1, -1)
    b1_1 = l1_b1.reshape(1, -1)
    bias0 = l0_bias.reshape(1, -1)
    bias1 = l1_bias.reshape(1, -1)

    zeros128 = jnp.zeros((n, 128), jnp.float32)

    gx = _sc_gather(x, src)
    msg0 = _c0(edge_attr, gx, l0_w1, b1_0, w2p0, b2m0)
    parts0 = _sc_scatter_add(msg0, dst3, zeros128)
    h0 = _r(parts0, x, l0_root, bias0)
    gh = _sc_gather(h0, src)
    msg1 = _c1(edge_attr, gh, l1_w1, b1_1, w2_1, r1, b2m1)
    parts1 = _sc_scatter_add(msg1, dst3, zeros128)
    out = _readout(parts1, h0, l1_root, bias1,
                   batch.reshape(n, 1), n_graphs,
                   fc1_w, fc1_b.reshape(1, -1),
                   fc2_w, fc2_b.reshape(1, -1),
                   fc3_w, fc3_b.reshape(1, -1))
    return out


# speedup vs baseline: 1.8147x; 1.0652x over previous
import functools

import jax
import jax.numpy as jnp
from jax import lax
from jax.experimental import pallas as pl
from jax.experimental.pallas import tpu as pltpu
from jax.experimental.pallas import tpu_sc as plsc

NC = 2
NS = 16
NW = NC * NS


def _elu(v):
    return jnp.where(v > 0, v, jnp.exp(v) - 1.0)


def _sc_gather(table, idx):
    n_rows, d = table.shape
    e = idx.shape[0]
    per_w = e // NW
    assert per_w * NW == e and per_w % 8 == 0
    chunk = 128
    n_full = per_w // chunk
    tail = per_w - n_full * chunk

    mesh = plsc.VectorSubcoreMesh(core_axis_name="c", subcore_axis_name="s")

    @functools.partial(
        pl.kernel,
        out_type=jax.ShapeDtypeStruct((e, d), table.dtype),
        mesh=mesh,
        scratch_types=[
            pltpu.VMEM((per_w,), jnp.int32),
            pltpu.VMEM((chunk, d), table.dtype),
            pltpu.SemaphoreType.DMA,
        ],
    )
    def k(table_hbm, idx_hbm, out_hbm, idx_v, rows_v, sem):
        wid = lax.axis_index("s") * NC + lax.axis_index("c")
        base = wid * per_w
        pltpu.sync_copy(idx_hbm.at[pl.ds(base, per_w)], idx_v)

        @pl.loop(0, n_full)
        def _(j):
            pltpu.async_copy(
                table_hbm.at[idx_v.at[pl.ds(j * chunk, chunk)]], rows_v, sem
            ).wait()
            pltpu.sync_copy(rows_v, out_hbm.at[pl.ds(base + j * chunk, chunk)])

        if tail:
            pltpu.async_copy(
                table_hbm.at[idx_v.at[pl.ds(n_full * chunk, tail)]],
                rows_v.at[pl.ds(0, tail)], sem,
            ).wait()
            pltpu.sync_copy(
                rows_v.at[pl.ds(0, tail)],
                out_hbm.at[pl.ds(base + n_full * chunk, tail)],
            )

    return k(table, idx)


def _sc_scatter_add(msg, dst3, zeros_nd):
    e, d = msg.shape
    n_nodes = zeros_nd.shape[0]
    nw, n_chunk, chunk = dst3.shape
    assert nw == NW and n_chunk * chunk * NW == e and chunk % 8 == 0
    per_w = n_chunk * chunk
    stripe = (n_nodes // NS) // 8 * 8
    rem = n_nodes - stripe * NS
    assert rem % 8 == 0

    mesh = plsc.VectorSubcoreMesh(core_axis_name="c", subcore_axis_name="s")

    @functools.partial(
        pl.kernel,
        out_type=jax.ShapeDtypeStruct((NC, n_nodes, d), msg.dtype),
        mesh=mesh,
        scratch_types=[
            pltpu.VMEM((n_chunk, chunk), jnp.int32),
            pltpu.VMEM((chunk, d), msg.dtype),
            pltpu.VMEM_SHARED((n_nodes, d), msg.dtype),
        ],
    )
    def k(msg_hbm, dst_hbm, zeros_hbm, out_hbm, idx_v, msg_v, acc_sh):
        cid = lax.axis_index("c")
        sid = lax.axis_index("s")
        wid = sid * NC + cid
        base = wid * per_w

        pltpu.sync_copy(
            zeros_hbm.at[pl.ds(sid * stripe, stripe)],
            acc_sh.at[pl.ds(sid * stripe, stripe)],
        )
        if rem:
            @pl.when(sid == NS - 1)
            def _():
                pltpu.sync_copy(
                    zeros_hbm.at[pl.ds(NS * stripe, rem)],
                    acc_sh.at[pl.ds(NS * stripe, rem)],
                )
        pltpu.sync_copy(dst_hbm.at[wid], idx_v)
        plsc.subcore_barrier()

        @pl.loop(0, n_chunk)
        def _(j):
            pltpu.sync_copy(msg_hbm.at[pl.ds(base + j * chunk, chunk)], msg_v)
            pltpu.sync_copy(msg_v, acc_sh.at[idx_v.at[j]], add=True)

        plsc.subcore_barrier()
        pltpu.sync_copy(
            acc_sh.at[pl.ds(sid * stripe, stripe)],
            out_hbm.at[cid, pl.ds(sid * stripe, stripe)],
        )
        if rem:
            @pl.when(sid == NS - 1)
            def _():
                pltpu.sync_copy(
                    acc_sh.at[pl.ds(NS * stripe, rem)],
                    out_hbm.at[cid, pl.ds(NS * stripe, rem)],
                )

    return k(msg, dst3, zeros_nd)


EB = 256


def _expand(v_bf16, r_ref):
    return jnp.dot(v_bf16, r_ref[...], preferred_element_type=jnp.float32)


def _fold_contract(wp, vr, width, d_out):
    p = wp * vr
    while width > d_out:
        width //= 2
        p = p[:, :width] + p[:, width:2 * width]
    return p


def _c0_body(ea_ref, gx_ref, w1_ref, b1_ref, w2p_ref, b2m_ref, out_ref):
    h = jnp.maximum(
        jnp.dot(ea_ref[...].astype(jnp.bfloat16), w1_ref[...],
                preferred_element_type=jnp.float32) + b1_ref[...], 0.0)
    wpb = jnp.dot(h.astype(jnp.bfloat16), w2p_ref[...],
                  preferred_element_type=jnp.float32
                  ).astype(jnp.bfloat16).astype(jnp.float32)
    gx = gx_ref[...]
    gxb = gx.astype(jnp.bfloat16).astype(jnp.float32)
    d_out = w2p_ref.shape[1] // 128
    cols = []
    for o in range(d_out):
        blk = wpb[:, o * 128:(o + 1) * 128]
        cols.append(jnp.sum(blk * gxb, axis=1, keepdims=True))
    p = jnp.concatenate(cols, axis=1) + jnp.dot(
        gxb.astype(jnp.bfloat16), b2m_ref[...],
        preferred_element_type=jnp.float32)
    out_ref[:, :d_out] = p


def _c0(ea, gx, w1, b1, w2p, b2m):
    e = ea.shape[0]
    grid = (e // EB,)
    return pl.pallas_call(
        _c0_body,
        grid=grid,
        in_specs=[
            pl.BlockSpec((EB, ea.shape[1]), lambda i: (i, 0)),
            pl.BlockSpec((EB, gx.shape[1]), lambda i: (i, 0)),
            pl.BlockSpec(w1.shape, lambda i: (0, 0)),
            pl.BlockSpec(b1.shape, lambda i: (0, 0)),
            pl.BlockSpec(w2p.shape, lambda i: (0, 0)),
            pl.BlockSpec(b2m.shape, lambda i: (0, 0)),
        ],
        out_specs=pl.BlockSpec((EB, 128), lambda i: (i, 0)),
        out_shape=jax.ShapeDtypeStruct((e, 128), jnp.float32),
    )(ea, gx, w1, b1, w2p, b2m)


def _c1_body(ea_ref, gh_ref, w1_ref, b1_ref, w2_ref, r_ref, b2m_ref, out_ref):
    d_in = r_ref.shape[0]
    d_out = w2_ref.shape[1] // d_in
    h = jnp.maximum(
        jnp.dot(ea_ref[...].astype(jnp.bfloat16), w1_ref[...],
                preferred_element_type=jnp.float32) + b1_ref[...], 0.0)
    wb = jnp.dot(h.astype(jnp.bfloat16), w2_ref[...],
                 preferred_element_type=jnp.float32
                 ).astype(jnp.bfloat16).astype(jnp.float32)
    ghb = gh_ref[...][:, :d_in].astype(jnp.bfloat16)
    ghr = _expand(ghb, r_ref)
    p = _fold_contract(wb, ghr, d_in * d_out, d_out)
    p = p + jnp.dot(ghb, b2m_ref[...], preferred_element_type=jnp.float32)
    out_ref[:, :d_out] = p


def _c1(ea, gh, w1, b1, w2, rmat, b2m):
    e = ea.shape[0]
    grid = (e // EB,)
    return pl.pallas_call(
        _c1_body,
        grid=grid,
        in_specs=[
            pl.BlockSpec((EB, ea.shape[1]), lambda i: (i, 0)),
            pl.BlockSpec((EB, gh.shape[1]), lambda i: (i, 0)),
            pl.BlockSpec(w1.shape, lambda i: (0, 0)),
            pl.BlockSpec(b1.shape, lambda i: (0, 0)),
            pl.BlockSpec(w2.shape, lambda i: (0, 0)),
            pl.BlockSpec(rmat.shape, lambda i: (0, 0)),
            pl.BlockSpec(b2m.shape, lambda i: (0, 0)),
        ],
        out_specs=pl.BlockSpec((EB, 128), lambda i: (i, 0)),
        out_shape=jax.ShapeDtypeStruct((e, 128), jnp.float32),
    )(ea, gh, w1, b1, w2, rmat, b2m)


def _r_body(p_ref, x_ref, root_ref, bias_ref, out_ref):
    d = root_ref.shape[1]
    agg = p_ref[0][:, :d] + p_ref[1][:, :d]
    xr = jnp.dot(x_ref[...].astype(jnp.bfloat16), root_ref[...],
                 preferred_element_type=jnp.float32)
    h = _elu(agg + xr + bias_ref[...])
    pad = out_ref.shape[1] - h.shape[1]
    out_ref[...] = jnp.concatenate(
        [h, jnp.zeros((h.shape[0], pad), jnp.float32)], axis=1)


def _r(parts, x, root, bias):
    n = parts.shape[1]
    return pl.pallas_call(
        _r_body,
        in_specs=[
            pl.BlockSpec(parts.shape, lambda: (0, 0, 0)),
            pl.BlockSpec(x.shape, lambda: (0, 0)),
            pl.BlockSpec(root.shape, lambda: (0, 0)),
            pl.BlockSpec(bias.shape, lambda: (0, 0)),
        ],
        out_specs=pl.BlockSpec((n, 128), lambda: (0, 0)),
        out_shape=jax.ShapeDtypeStruct((n, 128), jnp.float32),
    )(parts, x, root, bias)


def _readout_body(p_ref, h0_ref, root_ref, bias_ref, batch_ref,
                  fc1w_ref, fc1b_ref, fc2w_ref, fc2b_ref, fc3w_ref, fc3b_ref,
                  out_ref):
    n = h0_ref.shape[0]
    g_count = out_ref.shape[0]
    d = root_ref.shape[1]
    h1 = _elu(p_ref[0][:, :d] + p_ref[1][:, :d]
              + jnp.dot(h0_ref[...][:, :root_ref.shape[0]].astype(
                  jnp.bfloat16), root_ref[...],
                  preferred_element_type=jnp.float32)
              + bias_ref[...])
    gids = lax.broadcasted_iota(jnp.int32, (n, g_count), 1)
    oneh = (batch_ref[...] == gids).astype(jnp.float32)
    gsum = lax.dot_general(oneh, h1, (((0,), (0,)), ((), ())),
                           preferred_element_type=jnp.float32,
                           precision=jax.lax.Precision.HIGHEST)
    counts = jnp.sum(oneh, axis=0)
    g = gsum / jnp.clip(counts, 1.0)[:, None]
    g = _elu(jnp.dot(g.astype(jnp.bfloat16), fc1w_ref[...],
                     preferred_element_type=jnp.float32) + fc1b_ref[...])
    g = _elu(jnp.dot(g.astype(jnp.bfloat16), fc2w_ref[...],
                     preferred_element_type=jnp.float32) + fc2b_ref[...])
    out_ref[...] = (jnp.dot(g.astype(jnp.bfloat16), fc3w_ref[...],
                            preferred_element_type=jnp.float32)
                    + fc3b_ref[...])


def _readout(parts, h0, root, bias, batch2, n_graphs,
             fc1w, fc1b, fc2w, fc2b, fc3w, fc3b):
    n = h0.shape[0]
    full = lambda a: pl.BlockSpec(a.shape, lambda: tuple(0 for _ in a.shape))
    out = pl.pallas_call(
        _readout_body,
        in_specs=[full(parts), full(h0), full(root), full(bias), full(batch2),
                  full(fc1w), full(fc1b), full(fc2w), full(fc2b), full(fc3w),
                  full(fc3b)],
        out_specs=pl.BlockSpec((n_graphs, 1), lambda: (0, 0)),
        out_shape=jax.ShapeDtypeStruct((n_graphs, 1), jnp.float32),
    )(parts, h0, root, bias, batch2, fc1w, fc1b, fc2w, fc2b, fc3w, fc3b)
    return out.reshape(-1)


def kernel(x, edge_index, edge_attr, batch,
           l0_w1, l0_b1, l0_w2, l0_b2, l0_root, l0_bias,
           l1_w1, l1_b1, l1_w2, l1_b2, l1_root, l1_bias,
           fc1_w, fc1_b, fc2_w, fc2_b, fc3_w, fc3_b):
    n, d = x.shape
    e = edge_attr.shape[0]
    d0 = l0_root.shape[1]
    d1 = l1_root.shape[1]
    n_graphs = 64

    src = edge_index[0]
    dst = edge_index[1]
    per_w = e // NW
    chunk = 40
    n_chunk = per_w // chunk
    dst3 = dst.reshape(NW, n_chunk, chunk)

    bf = jnp.bfloat16
    w2p0 = l0_w2.reshape(d, d, d0).transpose(0, 2, 1).reshape(
        d, d * d0).astype(bf)
    b2m0 = l0_b2.reshape(d, d0).astype(bf)
    w2_1 = l1_w2.astype(bf)
    b2m1 = l1_b2.reshape(d0, d1).astype(bf)
    r1 = jnp.kron(jnp.eye(d0, dtype=jnp.float32),
                  jnp.ones((1, d1), jnp.float32)).astype(bf)
    w1_0 = l0_w1.astype(bf)
    w1_1 = l1_w1.astype(bf)
    root0 = l0_root.astype(bf)
    root1 = l1_root.astype(bf)
    fc1wb = fc1_w.astype(bf)
    fc2wb = fc2_w.astype(bf)
    fc3wb = fc3_w.astype(bf)
    b1_0 = l0_b1.reshape(1, -1)
    b1_1 = l1_b1.reshape(1, -1)
    bias0 = l0_bias.reshape(1, -1)
    bias1 = l1_bias.reshape(1, -1)

    zeros128 = jnp.zeros((n, 128), jnp.float32)

    gx = _sc_gather(x, src)
    msg0 = _c0(edge_attr, gx, w1_0, b1_0, w2p0, b2m0)
    parts0 = _sc_scatter_add(msg0, dst3, zeros128)
    h0 = _r(parts0, x, root0, bias0)
    gh = _sc_gather(h0, src)
    msg1 = _c1(edge_attr, gh, w1_1, b1_1, w2_1, r1, b2m1)
    parts1 = _sc_scatter_add(msg1, dst3, zeros128)
    out = _readout(parts1, h0, root1, bias1,
                   batch.reshape(n, 1), n_graphs,
                   fc1wb, fc1_b.reshape(1, -1),
                   fc2wb, fc2_b.reshape(1, -1),
                   fc3wb, fc3_b.reshape(1, -1))
    return out


# speedup vs baseline: 1.9294x; 1.0632x over previous
import functools

import jax
import jax.numpy as jnp
from jax import lax
from jax.experimental import pallas as pl
from jax.experimental.pallas import tpu as pltpu
from jax.experimental.pallas import tpu_sc as plsc

NC = 2
NS = 16
NW = NC * NS


def _elu(v):
    return jnp.where(v > 0, v, jnp.exp(v) - 1.0)


def _sc_gather(table, idx):
    n_rows, d = table.shape
    e = idx.shape[0]
    per_w = e // NW
    assert per_w * NW == e and per_w % 8 == 0
    chunk = 128
    n_full = per_w // chunk
    tail = per_w - n_full * chunk

    mesh = plsc.VectorSubcoreMesh(core_axis_name="c", subcore_axis_name="s")

    @functools.partial(
        pl.kernel,
        out_type=jax.ShapeDtypeStruct((e, d), table.dtype),
        mesh=mesh,
        scratch_types=[
            pltpu.VMEM((per_w,), jnp.int32),
            pltpu.VMEM((chunk, d), table.dtype),
            pltpu.SemaphoreType.DMA,
        ],
    )
    def k(table_hbm, idx_hbm, out_hbm, idx_v, rows_v, sem):
        wid = lax.axis_index("s") * NC + lax.axis_index("c")
        base = wid * per_w
        pltpu.sync_copy(idx_hbm.at[pl.ds(base, per_w)], idx_v)

        @pl.loop(0, n_full)
        def _(j):
            pltpu.async_copy(
                table_hbm.at[idx_v.at[pl.ds(j * chunk, chunk)]], rows_v, sem
            ).wait()
            pltpu.sync_copy(rows_v, out_hbm.at[pl.ds(base + j * chunk, chunk)])

        if tail:
            pltpu.async_copy(
                table_hbm.at[idx_v.at[pl.ds(n_full * chunk, tail)]],
                rows_v.at[pl.ds(0, tail)], sem,
            ).wait()
            pltpu.sync_copy(
                rows_v.at[pl.ds(0, tail)],
                out_hbm.at[pl.ds(base + n_full * chunk, tail)],
            )

    return k(table, idx)


def _sc_scatter_add(msg, dst3, zeros_nd):
    e, d = msg.shape
    n_nodes = zeros_nd.shape[0]
    nw, n_chunk, chunk = dst3.shape
    assert nw == NW and n_chunk * chunk * NW == e and chunk % 8 == 0
    per_w = n_chunk * chunk
    stripe = (n_nodes // NS) // 8 * 8
    rem = n_nodes - stripe * NS
    assert rem % 8 == 0

    mesh = plsc.VectorSubcoreMesh(core_axis_name="c", subcore_axis_name="s")

    @functools.partial(
        pl.kernel,
        out_type=jax.ShapeDtypeStruct((NC, n_nodes, d), msg.dtype),
        mesh=mesh,
        scratch_types=[
            pltpu.VMEM((n_chunk, chunk), jnp.int32),
            pltpu.VMEM((chunk, d), msg.dtype),
            pltpu.VMEM((chunk, d), msg.dtype),
            pltpu.VMEM_SHARED((n_nodes, d), msg.dtype),
            pltpu.SemaphoreType.DMA,
            pltpu.SemaphoreType.DMA,
        ],
    )
    def k(msg_hbm, dst_hbm, zeros_hbm, out_hbm, idx_v, msg_a, msg_b, acc_sh,
          sem_a, sem_b):
        cid = lax.axis_index("c")
        sid = lax.axis_index("s")
        wid = sid * NC + cid
        base = wid * per_w

        pltpu.sync_copy(
            zeros_hbm.at[pl.ds(sid * stripe, stripe)],
            acc_sh.at[pl.ds(sid * stripe, stripe)],
        )
        if rem:
            @pl.when(sid == NS - 1)
            def _():
                pltpu.sync_copy(
                    zeros_hbm.at[pl.ds(NS * stripe, rem)],
                    acc_sh.at[pl.ds(NS * stripe, rem)],
                )
        pltpu.sync_copy(dst_hbm.at[wid], idx_v)
        plsc.subcore_barrier()

        def _load(j, buf, sem):
            pltpu.async_copy(
                msg_hbm.at[pl.ds(base + j * chunk, chunk)], buf, sem)

        def _wait(j, buf, sem):
            pltpu.make_async_copy(
                msg_hbm.at[pl.ds(base + j * chunk, chunk)], buf, sem).wait()

        assert n_chunk % 2 == 1
        _load(0, msg_a, sem_a)

        @pl.loop(0, n_chunk - 1, step=2)
        def _(j):
            _load(j + 1, msg_b, sem_b)
            _wait(j, msg_a, sem_a)
            pltpu.sync_copy(msg_a, acc_sh.at[idx_v.at[j]], add=True)
            _load(j + 2, msg_a, sem_a)
            _wait(j + 1, msg_b, sem_b)
            pltpu.sync_copy(msg_b, acc_sh.at[idx_v.at[j + 1]], add=True)

        _wait(n_chunk - 1, msg_a, sem_a)
        pltpu.sync_copy(msg_a, acc_sh.at[idx_v.at[n_chunk - 1]], add=True)

        plsc.subcore_barrier()
        pltpu.sync_copy(
            acc_sh.at[pl.ds(sid * stripe, stripe)],
            out_hbm.at[cid, pl.ds(sid * stripe, stripe)],
        )
        if rem:
            @pl.when(sid == NS - 1)
            def _():
                pltpu.sync_copy(
                    acc_sh.at[pl.ds(NS * stripe, rem)],
                    out_hbm.at[cid, pl.ds(NS * stripe, rem)],
                )

    return k(msg, dst3, zeros_nd)


EB = 256


def _expand(v_bf16, r_ref):
    return jnp.dot(v_bf16, r_ref[...], preferred_element_type=jnp.float32)


def _fold_contract(wp, vr, width, d_out):
    p = wp * vr
    while width > d_out:
        width //= 2
        p = p[:, :width] + p[:, width:2 * width]
    return p


def _c0_body(ea_ref, gx_ref, w1_ref, b1_ref, w2p_ref, b2m_ref, out_ref):
    h = jnp.maximum(
        jnp.dot(ea_ref[...].astype(jnp.bfloat16), w1_ref[...],
                preferred_element_type=jnp.float32) + b1_ref[...], 0.0)
    wpb = jnp.dot(h.astype(jnp.bfloat16), w2p_ref[...],
                  preferred_element_type=jnp.float32
                  ).astype(jnp.bfloat16).astype(jnp.float32)
    gx = gx_ref[...]
    gxb = gx.astype(jnp.bfloat16).astype(jnp.float32)
    d_out = w2p_ref.shape[1] // 128
    cols = []
    for o in range(d_out):
        blk = wpb[:, o * 128:(o + 1) * 128]
        cols.append(jnp.sum(blk * gxb, axis=1, keepdims=True))
    p = jnp.concatenate(cols, axis=1) + jnp.dot(
        gxb.astype(jnp.bfloat16), b2m_ref[...],
        preferred_element_type=jnp.float32)
    out_ref[:, :d_out] = p


def _c0(ea, gx, w1, b1, w2p, b2m):
    e = ea.shape[0]
    grid = (e // EB,)
    return pl.pallas_call(
        _c0_body,
        grid=grid,
        in_specs=[
            pl.BlockSpec((EB, ea.shape[1]), lambda i: (i, 0)),
            pl.BlockSpec((EB, gx.shape[1]), lambda i: (i, 0)),
            pl.BlockSpec(w1.shape, lambda i: (0, 0)),
            pl.BlockSpec(b1.shape, lambda i: (0, 0)),
            pl.BlockSpec(w2p.shape, lambda i: (0, 0)),
            pl.BlockSpec(b2m.shape, lambda i: (0, 0)),
        ],
        out_specs=pl.BlockSpec((EB, 128), lambda i: (i, 0)),
        out_shape=jax.ShapeDtypeStruct((e, 128), jnp.float32),
    )(ea, gx, w1, b1, w2p, b2m)


def _c1_body(ea_ref, gh_ref, w1_ref, b1_ref, w2_ref, r_ref, b2m_ref, out_ref):
    d_in = r_ref.shape[0]
    d_out = w2_ref.shape[1] // d_in
    h = jnp.maximum(
        jnp.dot(ea_ref[...].astype(jnp.bfloat16), w1_ref[...],
                preferred_element_type=jnp.float32) + b1_ref[...], 0.0)
    wb = jnp.dot(h.astype(jnp.bfloat16), w2_ref[...],
                 preferred_element_type=jnp.float32
                 ).astype(jnp.bfloat16).astype(jnp.float32)
    ghb = gh_ref[...][:, :d_in].astype(jnp.bfloat16)
    ghr = _expand(ghb, r_ref)
    p = _fold_contract(wb, ghr, d_in * d_out, d_out)
    p = p + jnp.dot(ghb, b2m_ref[...], preferred_element_type=jnp.float32)
    out_ref[:, :d_out] = p


def _c1(ea, gh, w1, b1, w2, rmat, b2m):
    e = ea.shape[0]
    grid = (e // EB,)
    return pl.pallas_call(
        _c1_body,
        grid=grid,
        in_specs=[
            pl.BlockSpec((EB, ea.shape[1]), lambda i: (i, 0)),
            pl.BlockSpec((EB, gh.shape[1]), lambda i: (i, 0)),
            pl.BlockSpec(w1.shape, lambda i: (0, 0)),
            pl.BlockSpec(b1.shape, lambda i: (0, 0)),
            pl.BlockSpec(w2.shape, lambda i: (0, 0)),
            pl.BlockSpec(rmat.shape, lambda i: (0, 0)),
            pl.BlockSpec(b2m.shape, lambda i: (0, 0)),
        ],
        out_specs=pl.BlockSpec((EB, 128), lambda i: (i, 0)),
        out_shape=jax.ShapeDtypeStruct((e, 128), jnp.float32),
    )(ea, gh, w1, b1, w2, rmat, b2m)


def _r_body(p_ref, x_ref, root_ref, bias_ref, out_ref):
    d = root_ref.shape[1]
    agg = p_ref[0][:, :d] + p_ref[1][:, :d]
    xr = jnp.dot(x_ref[...].astype(jnp.bfloat16), root_ref[...],
                 preferred_element_type=jnp.float32)
    h = _elu(agg + xr + bias_ref[...])
    pad = out_ref.shape[1] - h.shape[1]
    out_ref[...] = jnp.concatenate(
        [h, jnp.zeros((h.shape[0], pad), jnp.float32)], axis=1)


def _r(parts, x, root, bias):
    n = parts.shape[1]
    return pl.pallas_call(
        _r_body,
        in_specs=[
            pl.BlockSpec(parts.shape, lambda: (0, 0, 0)),
            pl.BlockSpec(x.shape, lambda: (0, 0)),
            pl.BlockSpec(root.shape, lambda: (0, 0)),
            pl.BlockSpec(bias.shape, lambda: (0, 0)),
        ],
        out_specs=pl.BlockSpec((n, 128), lambda: (0, 0)),
        out_shape=jax.ShapeDtypeStruct((n, 128), jnp.float32),
    )(parts, x, root, bias)


def _readout_body(p_ref, h0_ref, root_ref, bias_ref, batch_ref,
                  fc1w_ref, fc1b_ref, fc2w_ref, fc2b_ref, fc3w_ref, fc3b_ref,
                  out_ref):
    n = h0_ref.shape[0]
    g_count = out_ref.shape[0]
    d = root_ref.shape[1]
    h1 = _elu(p_ref[0][:, :d] + p_ref[1][:, :d]
              + jnp.dot(h0_ref[...][:, :root_ref.shape[0]].astype(
                  jnp.bfloat16), root_ref[...],
                  preferred_element_type=jnp.float32)
              + bias_ref[...])
    gids = lax.broadcasted_iota(jnp.int32, (n, g_count), 1)
    oneh = (batch_ref[...] == gids).astype(jnp.float32)
    gsum = lax.dot_general(oneh, h1, (((0,), (0,)), ((), ())),
                           preferred_element_type=jnp.float32,
                           precision=jax.lax.Precision.HIGHEST)
    counts = jnp.sum(oneh, axis=0)
    g = gsum / jnp.clip(counts, 1.0)[:, None]
    g = _elu(jnp.dot(g.astype(jnp.bfloat16), fc1w_ref[...],
                     preferred_element_type=jnp.float32) + fc1b_ref[...])
    g = _elu(jnp.dot(g.astype(jnp.bfloat16), fc2w_ref[...],
                     preferred_element_type=jnp.float32) + fc2b_ref[...])
    out_ref[...] = (jnp.dot(g.astype(jnp.bfloat16), fc3w_ref[...],
                            preferred_element_type=jnp.float32)
                    + fc3b_ref[...])


def _readout(parts, h0, root, bias, batch2, n_graphs,
             fc1w, fc1b, fc2w, fc2b, fc3w, fc3b):
    n = h0.shape[0]
    full = lambda a: pl.BlockSpec(a.shape, lambda: tuple(0 for _ in a.shape))
    out = pl.pallas_call(
        _readout_body,
        in_specs=[full(parts), full(h0), full(root), full(bias), full(batch2),
                  full(fc1w), full(fc1b), full(fc2w), full(fc2b), full(fc3w),
                  full(fc3b)],
        out_specs=pl.BlockSpec((n_graphs, 1), lambda: (0, 0)),
        out_shape=jax.ShapeDtypeStruct((n_graphs, 1), jnp.float32),
    )(parts, h0, root, bias, batch2, fc1w, fc1b, fc2w, fc2b, fc3w, fc3b)
    return out.reshape(-1)


def kernel(x, edge_index, edge_attr, batch,
           l0_w1, l0_b1, l0_w2, l0_b2, l0_root, l0_bias,
           l1_w1, l1_b1, l1_w2, l1_b2, l1_root, l1_bias,
           fc1_w, fc1_b, fc2_w, fc2_b, fc3_w, fc3_b):
    n, d = x.shape
    e = edge_attr.shape[0]
    d0 = l0_root.shape[1]
    d1 = l1_root.shape[1]
    n_graphs = 64

    src = edge_index[0]
    dst = edge_index[1]
    per_w = e // NW
    chunk = 40
    n_chunk = per_w // chunk
    dst3 = dst.reshape(NW, n_chunk, chunk)

    bf = jnp.bfloat16
    w2p0 = l0_w2.reshape(d, d, d0).transpose(0, 2, 1).reshape(
        d, d * d0).astype(bf)
    b2m0 = l0_b2.reshape(d, d0).astype(bf)
    w2_1 = l1_w2.astype(bf)
    b2m1 = l1_b2.reshape(d0, d1).astype(bf)
    r1 = jnp.kron(jnp.eye(d0, dtype=jnp.float32),
                  jnp.ones((1, d1), jnp.float32)).astype(bf)
    w1_0 = l0_w1.astype(bf)
    w1_1 = l1_w1.astype(bf)
    root0 = l0_root.astype(bf)
    root1 = l1_root.astype(bf)
    fc1wb = fc1_w.astype(bf)
    fc2wb = fc2_w.astype(bf)
    fc3wb = fc3_w.astype(bf)
    b1_0 = l0_b1.reshape(1, -1)
    b1_1 = l1_b1.reshape(1, -1)
    bias0 = l0_bias.reshape(1, -1)
    bias1 = l1_bias.reshape(1, -1)

    zeros128 = jnp.zeros((n, 128), jnp.float32)

    gx = _sc_gather(x, src)
    msg0 = _c0(edge_attr, gx, w1_0, b1_0, w2p0, b2m0)
    parts0 = _sc_scatter_add(msg0, dst3, zeros128)
    h0 = _r(parts0, x, root0, bias0)
    gh = _sc_gather(h0, src)
    msg1 = _c1(edge_attr, gh, w1_1, b1_1, w2_1, r1, b2m1)
    parts1 = _sc_scatter_add(msg1, dst3, zeros128)
    out = _readout(parts1, h0, root1, bias1,
                   batch.reshape(n, 1), n_graphs,
                   fc1wb, fc1_b.reshape(1, -1),
                   fc2wb, fc2_b.reshape(1, -1),
                   fc3wb, fc3_b.reshape(1, -1))
    return out


# speedup vs baseline: 1.9660x; 1.0189x over previous
import functools

import jax
import jax.numpy as jnp
from jax import lax
from jax.experimental import pallas as pl
from jax.experimental.pallas import tpu as pltpu
from jax.experimental.pallas import tpu_sc as plsc

NC = 2
NS = 16
NW = NC * NS


def _elu(v):
    return jnp.where(v > 0, v, jnp.exp(v) - 1.0)


def _sc_gather(table, idx):
    n_rows, d = table.shape
    e = idx.shape[0]
    per_w = e // NW
    assert per_w * NW == e and per_w % 8 == 0
    chunk = 128
    n_full = per_w // chunk
    tail = per_w - n_full * chunk

    mesh = plsc.VectorSubcoreMesh(core_axis_name="c", subcore_axis_name="s")

    @functools.partial(
        pl.kernel,
        out_type=jax.ShapeDtypeStruct((e, d), table.dtype),
        mesh=mesh,
        scratch_types=[
            pltpu.VMEM((per_w,), jnp.int32),
            pltpu.VMEM((chunk, d), table.dtype),
            pltpu.VMEM((chunk, d), table.dtype),
            pltpu.SemaphoreType.DMA,
            pltpu.SemaphoreType.DMA,
        ],
    )
    def k(table_hbm, idx_hbm, out_hbm, idx_v, rows_a, rows_b, sem_a, sem_b):
        wid = lax.axis_index("s") * NC + lax.axis_index("c")
        base = wid * per_w
        pltpu.sync_copy(idx_hbm.at[pl.ds(base, per_w)], idx_v)

        def _start(j, buf, sem):
            pltpu.async_copy(
                table_hbm.at[idx_v.at[pl.ds(j * chunk, chunk)]], buf, sem)

        def _wait(j, buf, sem):
            pltpu.make_async_copy(
                table_hbm.at[idx_v.at[pl.ds(j * chunk, chunk)]], buf,
                sem).wait()

        def _out(j, buf):
            pltpu.sync_copy(buf, out_hbm.at[pl.ds(base + j * chunk, chunk)])

        assert n_full % 2 == 1
        _start(0, rows_a, sem_a)

        @pl.loop(0, n_full - 1, step=2)
        def _(j):
            _start(j + 1, rows_b, sem_b)
            _wait(j, rows_a, sem_a)
            _out(j, rows_a)
            _start(j + 2, rows_a, sem_a)
            _wait(j + 1, rows_b, sem_b)
            _out(j + 1, rows_b)

        _wait(n_full - 1, rows_a, sem_a)
        _out(n_full - 1, rows_a)

        if tail:
            pltpu.async_copy(
                table_hbm.at[idx_v.at[pl.ds(n_full * chunk, tail)]],
                rows_b.at[pl.ds(0, tail)], sem_b,
            ).wait()
            pltpu.sync_copy(
                rows_b.at[pl.ds(0, tail)],
                out_hbm.at[pl.ds(base + n_full * chunk, tail)],
            )

    return k(table, idx)


def _sc_scatter_add(msg, dst3, zeros_nd):
    e, d = msg.shape
    n_nodes = zeros_nd.shape[0]
    nw, n_chunk, chunk = dst3.shape
    assert nw == NW and n_chunk * chunk * NW == e and chunk % 8 == 0
    per_w = n_chunk * chunk
    stripe = (n_nodes // NS) // 8 * 8
    rem = n_nodes - stripe * NS
    assert rem % 8 == 0

    mesh = plsc.VectorSubcoreMesh(core_axis_name="c", subcore_axis_name="s")

    @functools.partial(
        pl.kernel,
        out_type=jax.ShapeDtypeStruct((NC, n_nodes, d), msg.dtype),
        mesh=mesh,
        scratch_types=[
            pltpu.VMEM((n_chunk, chunk), jnp.int32),
            pltpu.VMEM((chunk, d), msg.dtype),
            pltpu.VMEM((chunk, d), msg.dtype),
            pltpu.VMEM_SHARED((n_nodes, d), msg.dtype),
            pltpu.SemaphoreType.DMA,
            pltpu.SemaphoreType.DMA,
        ],
    )
    def k(msg_hbm, dst_hbm, zeros_hbm, out_hbm, idx_v, msg_a, msg_b, acc_sh,
          sem_a, sem_b):
        cid = lax.axis_index("c")
        sid = lax.axis_index("s")
        wid = sid * NC + cid
        base = wid * per_w

        pltpu.sync_copy(
            zeros_hbm.at[pl.ds(sid * stripe, stripe)],
            acc_sh.at[pl.ds(sid * stripe, stripe)],
        )
        if rem:
            @pl.when(sid == NS - 1)
            def _():
                pltpu.sync_copy(
                    zeros_hbm.at[pl.ds(NS * stripe, rem)],
                    acc_sh.at[pl.ds(NS * stripe, rem)],
                )
        pltpu.sync_copy(dst_hbm.at[wid], idx_v)
        plsc.subcore_barrier()

        def _load(j, buf, sem):
            pltpu.async_copy(
                msg_hbm.at[pl.ds(base + j * chunk, chunk)], buf, sem)

        def _wait(j, buf, sem):
            pltpu.make_async_copy(
                msg_hbm.at[pl.ds(base + j * chunk, chunk)], buf, sem).wait()

        assert n_chunk % 2 == 1
        _load(0, msg_a, sem_a)

        @pl.loop(0, n_chunk - 1, step=2)
        def _(j):
            _load(j + 1, msg_b, sem_b)
            _wait(j, msg_a, sem_a)
            pltpu.sync_copy(msg_a, acc_sh.at[idx_v.at[j]], add=True)
            _load(j + 2, msg_a, sem_a)
            _wait(j + 1, msg_b, sem_b)
            pltpu.sync_copy(msg_b, acc_sh.at[idx_v.at[j + 1]], add=True)

        _wait(n_chunk - 1, msg_a, sem_a)
        pltpu.sync_copy(msg_a, acc_sh.at[idx_v.at[n_chunk - 1]], add=True)

        plsc.subcore_barrier()
        pltpu.sync_copy(
            acc_sh.at[pl.ds(sid * stripe, stripe)],
            out_hbm.at[cid, pl.ds(sid * stripe, stripe)],
        )
        if rem:
            @pl.when(sid == NS - 1)
            def _():
                pltpu.sync_copy(
                    acc_sh.at[pl.ds(NS * stripe, rem)],
                    out_hbm.at[cid, pl.ds(NS * stripe, rem)],
                )

    return k(msg, dst3, zeros_nd)


EB = 256


def _expand(v_bf16, r_ref):
    return jnp.dot(v_bf16, r_ref[...], preferred_element_type=jnp.float32)


def _fold_contract(wp, vr, width, d_out):
    p = wp * vr
    while width > d_out:
        width //= 2
        p = p[:, :width] + p[:, width:2 * width]
    return p


def _c0_body(ea_ref, gx_ref, w1_ref, b1_ref, w2p_ref, b2m_ref, out_ref):
    h = jnp.maximum(
        jnp.dot(ea_ref[...].astype(jnp.bfloat16), w1_ref[...],
                preferred_element_type=jnp.float32) + b1_ref[...], 0.0)
    wpb = jnp.dot(h.astype(jnp.bfloat16), w2p_ref[...],
                  preferred_element_type=jnp.float32
                  ).astype(jnp.bfloat16).astype(jnp.float32)
    gx = gx_ref[...]
    gxb = gx.astype(jnp.bfloat16).astype(jnp.float32)
    d_out = w2p_ref.shape[1] // 128
    cols = []
    for o in range(d_out):
        blk = wpb[:, o * 128:(o + 1) * 128]
        cols.append(jnp.sum(blk * gxb, axis=1, keepdims=True))
    p = jnp.concatenate(cols, axis=1) + jnp.dot(
        gxb.astype(jnp.bfloat16), b2m_ref[...],
        preferred_element_type=jnp.float32)
    out_ref[:, :d_out] = p


def _c0(ea, gx, w1, b1, w2p, b2m):
    e = ea.shape[0]
    grid = (e // EB,)
    return pl.pallas_call(
        _c0_body,
        grid=grid,
        in_specs=[
            pl.BlockSpec((EB, ea.shape[1]), lambda i: (i, 0)),
            pl.BlockSpec((EB, gx.shape[1]), lambda i: (i, 0)),
            pl.BlockSpec(w1.shape, lambda i: (0, 0)),
            pl.BlockSpec(b1.shape, lambda i: (0, 0)),
            pl.BlockSpec(w2p.shape, lambda i: (0, 0)),
            pl.BlockSpec(b2m.shape, lambda i: (0, 0)),
        ],
        out_specs=pl.BlockSpec((EB, 128), lambda i: (i, 0)),
        out_shape=jax.ShapeDtypeStruct((e, 128), jnp.float32),
    )(ea, gx, w1, b1, w2p, b2m)


def _c1_body(ea_ref, gh_ref, w1_ref, b1_ref, w2_ref, r_ref, b2m_ref, out_ref):
    d_in = r_ref.shape[0]
    d_out = w2_ref.shape[1] // d_in
    h = jnp.maximum(
        jnp.dot(ea_ref[...].astype(jnp.bfloat16), w1_ref[...],
                preferred_element_type=jnp.float32) + b1_ref[...], 0.0)
    wb = jnp.dot(h.astype(jnp.bfloat16), w2_ref[...],
                 preferred_element_type=jnp.float32
                 ).astype(jnp.bfloat16).astype(jnp.float32)
    ghb = gh_ref[...][:, :d_in].astype(jnp.bfloat16)
    ghr = _expand(ghb, r_ref)
    p = _fold_contract(wb, ghr, d_in * d_out, d_out)
    p = p + jnp.dot(ghb, b2m_ref[...], preferred_element_type=jnp.float32)
    out_ref[:, :d_out] = p


def _c1(ea, gh, w1, b1, w2, rmat, b2m):
    e = ea.shape[0]
    grid = (e // EB,)
    return pl.pallas_call(
        _c1_body,
        grid=grid,
        in_specs=[
            pl.BlockSpec((EB, ea.shape[1]), lambda i: (i, 0)),
            pl.BlockSpec((EB, gh.shape[1]), lambda i: (i, 0)),
            pl.BlockSpec(w1.shape, lambda i: (0, 0)),
            pl.BlockSpec(b1.shape, lambda i: (0, 0)),
            pl.BlockSpec(w2.shape, lambda i: (0, 0)),
            pl.BlockSpec(rmat.shape, lambda i: (0, 0)),
            pl.BlockSpec(b2m.shape, lambda i: (0, 0)),
        ],
        out_specs=pl.BlockSpec((EB, 128), lambda i: (i, 0)),
        out_shape=jax.ShapeDtypeStruct((e, 128), jnp.float32),
    )(ea, gh, w1, b1, w2, rmat, b2m)


def _r_body(p_ref, x_ref, root_ref, bias_ref, out_ref):
    d = root_ref.shape[1]
    agg = p_ref[0][:, :d] + p_ref[1][:, :d]
    xr = jnp.dot(x_ref[...].astype(jnp.bfloat16), root_ref[...],
                 preferred_element_type=jnp.float32)
    h = _elu(agg + xr + bias_ref[...])
    pad = out_ref.shape[1] - h.shape[1]
    out_ref[...] = jnp.concatenate(
        [h, jnp.zeros((h.shape[0], pad), jnp.float32)], axis=1)


def _r(parts, x, root, bias):
    n = parts.shape[1]
    return pl.pallas_call(
        _r_body,
        in_specs=[
            pl.BlockSpec(parts.shape, lambda: (0, 0, 0)),
            pl.BlockSpec(x.shape, lambda: (0, 0)),
            pl.BlockSpec(root.shape, lambda: (0, 0)),
            pl.BlockSpec(bias.shape, lambda: (0, 0)),
        ],
        out_specs=pl.BlockSpec((n, 128), lambda: (0, 0)),
        out_shape=jax.ShapeDtypeStruct((n, 128), jnp.float32),
    )(parts, x, root, bias)


def _readout_body(p_ref, h0_ref, root_ref, bias_ref, batch_ref,
                  fc1w_ref, fc1b_ref, fc2w_ref, fc2b_ref, fc3w_ref, fc3b_ref,
                  out_ref):
    n = h0_ref.shape[0]
    g_count = out_ref.shape[0]
    d = root_ref.shape[1]
    h1 = _elu(p_ref[0][:, :d] + p_ref[1][:, :d]
              + jnp.dot(h0_ref[...][:, :root_ref.shape[0]].astype(
                  jnp.bfloat16), root_ref[...],
                  preferred_element_type=jnp.float32)
              + bias_ref[...])
    gids = lax.broadcasted_iota(jnp.int32, (n, g_count), 1)
    oneh = (batch_ref[...] == gids).astype(jnp.float32)
    gsum = lax.dot_general(oneh, h1, (((0,), (0,)), ((), ())),
                           preferred_element_type=jnp.float32,
                           precision=jax.lax.Precision.HIGHEST)
    counts = jnp.sum(oneh, axis=0)
    g = gsum / jnp.clip(counts, 1.0)[:, None]
    g = _elu(jnp.dot(g.astype(jnp.bfloat16), fc1w_ref[...],
                     preferred_element_type=jnp.float32) + fc1b_ref[...])
    g = _elu(jnp.dot(g.astype(jnp.bfloat16), fc2w_ref[...],
                     preferred_element_type=jnp.float32) + fc2b_ref[...])
    out_ref[...] = (jnp.dot(g.astype(jnp.bfloat16), fc3w_ref[...],
                            preferred_element_type=jnp.float32)
                    + fc3b_ref[...])


def _readout(parts, h0, root, bias, batch2, n_graphs,
             fc1w, fc1b, fc2w, fc2b, fc3w, fc3b):
    n = h0.shape[0]
    full = lambda a: pl.BlockSpec(a.shape, lambda: tuple(0 for _ in a.shape))
    out = pl.pallas_call(
        _readout_body,
        in_specs=[full(parts), full(h0), full(root), full(bias), full(batch2),
                  full(fc1w), full(fc1b), full(fc2w), full(fc2b), full(fc3w),
                  full(fc3b)],
        out_specs=pl.BlockSpec((n_graphs, 1), lambda: (0, 0)),
        out_shape=jax.ShapeDtypeStruct((n_graphs, 1), jnp.float32),
    )(parts, h0, root, bias, batch2, fc1w, fc1b, fc2w, fc2b, fc3w, fc3b)
    return out.reshape(-1)


def kernel(x, edge_index, edge_attr, batch,
           l0_w1, l0_b1, l0_w2, l0_b2, l0_root, l0_bias,
           l1_w1, l1_b1, l1_w2, l1_b2, l1_root, l1_bias,
           fc1_w, fc1_b, fc2_w, fc2_b, fc3_w, fc3_b):
    n, d = x.shape
    e = edge_attr.shape[0]
    d0 = l0_root.shape[1]
    d1 = l1_root.shape[1]
    n_graphs = 64

    src = edge_index[0]
    dst = edge_index[1]
    per_w = e // NW
    chunk = 40
    n_chunk = per_w // chunk
    dst3 = dst.reshape(NW, n_chunk, chunk)

    bf = jnp.bfloat16
    w2p0 = l0_w2.reshape(d, d, d0).transpose(0, 2, 1).reshape(
        d, d * d0).astype(bf)
    b2m0 = l0_b2.reshape(d, d0).astype(bf)
    w2_1 = l1_w2.astype(bf)
    b2m1 = l1_b2.reshape(d0, d1).astype(bf)
    r1 = jnp.kron(jnp.eye(d0, dtype=jnp.float32),
                  jnp.ones((1, d1), jnp.float32)).astype(bf)
    w1_0 = l0_w1.astype(bf)
    w1_1 = l1_w1.astype(bf)
    root0 = l0_root.astype(bf)
    root1 = l1_root.astype(bf)
    fc1wb = fc1_w.astype(bf)
    fc2wb = fc2_w.astype(bf)
    fc3wb = fc3_w.astype(bf)
    b1_0 = l0_b1.reshape(1, -1)
    b1_1 = l1_b1.reshape(1, -1)
    bias0 = l0_bias.reshape(1, -1)
    bias1 = l1_bias.reshape(1, -1)

    zeros128 = jnp.zeros((n, 128), jnp.float32)

    gx = _sc_gather(x, src)
    msg0 = _c0(edge_attr, gx, w1_0, b1_0, w2p0, b2m0)
    parts0 = _sc_scatter_add(msg0, dst3, zeros128)
    h0 = _r(parts0, x, root0, bias0)
    gh = _sc_gather(h0, src)
    msg1 = _c1(edge_attr, gh, w1_1, b1_1, w2_1, r1, b2m1)
    parts1 = _sc_scatter_add(msg1, dst3, zeros128)
    out = _readout(parts1, h0, root1, bias1,
                   batch.reshape(n, 1), n_graphs,
                   fc1wb, fc1_b.reshape(1, -1),
                   fc2wb, fc2_b.reshape(1, -1),
                   fc3wb, fc3_b.reshape(1, -1))
    return out
